# baseline (speedup 1.0000x reference)
import jax
import jax.numpy as jnp
from jax import lax
from jax.experimental import pallas as pl
from jax.experimental.pallas import tpu as pltpu
from jax.experimental.pallas import tpu_sc as plsc

B = 8
SEG = 4096
SEQ = 2048
H = 1024
NC = 2
NS = 16
NW = NC * NS
CW = H // NW
CK = 128
CK3 = CK * 3
NCH = SEG // CK
CKSH = CK.bit_length() - 1


def _sc_body(emb_h, tok_h, pos_h, act_h,
             rows_v, posf_v, scale_v, acc, sems, isems, tok_cur, pos_cur):
    c = lax.axis_index("c")
    s = lax.axis_index("s")
    w = c * NS + s

    z16 = jnp.zeros((16,), jnp.float32)
    iota = lax.iota(jnp.int32, 16)

    tcur = [tok_cur.at[0], tok_cur.at[1]]
    pcur = [pos_cur.at[0], pos_cur.at[1]]

    for b in range(B):
        pltpu.sync_copy(pos_h.at[b], posf_v)
        vmidrow = posf_v[SEG // 2 // CK, pl.ds(0, 16)]
        vmid = jnp.full((16,), jnp.min(vmidrow), jnp.int32)

        @pl.loop(0, SEQ // 16)
        def _(q):
            p0 = q * 16 + iota
            lbs = []
            for delta in (0, 1):
                tgt = p0 + delta
                lt0 = vmid < tgt
                lo = jnp.where(lt0, SEG // 2 + 1, jnp.zeros((16,), jnp.int32))
                hi = jnp.where(lt0, jnp.full((16,), SEG, jnp.int32), SEG // 2)
                for _ in range(12):
                    mid = jnp.minimum((lo + hi) >> 1, SEG - 1)
                    v = plsc.load_gather(posf_v, [mid >> CKSH, mid & (CK - 1)])
                    lt = v < tgt
                    lo = jnp.where(lt, mid + 1, lo)
                    hi = jnp.where(lt, hi, mid)
                lbs.append(lo)
            cntf = (lbs[1] - lbs[0]).astype(jnp.float32)
            scale_v[pl.ds(q * 16, 16)] = (1.0 / 3.0) / jnp.maximum(cntf, 1.0)

        @pl.loop(0, SEQ, unroll=8)
        def _(p):
            acc[p, pl.ds(0, 16)] = z16
            acc[p, pl.ds(16, 16)] = z16

        def i_start(j, buf):
            pltpu.async_copy(tok_h.at[b, j], tcur[buf], isems.at[buf])
            pltpu.async_copy(pos_h.at[b, j], pcur[buf], isems.at[buf])

        def i_wait(j, buf):
            pltpu.make_async_copy(tok_h.at[b, j], tcur[buf],
                                  isems.at[buf]).wait()
            pltpu.make_async_copy(pos_h.at[b, j], pcur[buf],
                                  isems.at[buf]).wait()

        def add_w(buf):
            t = tcur[buf]

            @pl.loop(0, CK3 // 16, unroll=6)
            def _(i):
                t[pl.ds(i * 16, 16)] = t[pl.ds(i * 16, 16)] + w

        def g_start(buf):
            pltpu.async_copy(emb_h.at[tcur[buf]], rows_v.at[buf],
                             sems.at[buf])

        def g_wait(buf):
            pltpu.make_async_copy(emb_h.at[tcur[buf]], rows_v.at[buf],
                                  sems.at[buf]).wait()

        def accumulate(buf):
            rows = rows_v.at[buf]
            pv = pcur[buf]

            @pl.loop(0, CK, unroll=4)
            def _(si):
                psplat = plsc.load_gather(pv, [jnp.full((16,), si, jnp.int32)])
                ssplat = plsc.load_gather(scale_v, [psplat])
                r0 = si * 3
                v0 = (rows[r0, pl.ds(0, 16)] + rows[r0 + 1, pl.ds(0, 16)]
                      + rows[r0 + 2, pl.ds(0, 16)]) * ssplat
                plsc.addupdate_scatter(acc, [psplat, iota], v0)
                v1 = (rows[r0, pl.ds(16, 16)] + rows[r0 + 1, pl.ds(16, 16)]
                      + rows[r0 + 2, pl.ds(16, 16)]) * ssplat
                plsc.addupdate_scatter(acc, [psplat, iota + 16], v1)

        i_start(0, 0)
        i_start(1, 1)
        i_wait(0, 0)
        add_w(0)
        g_start(0)
        i_wait(1, 1)
        add_w(1)
        g_start(1)

        @pl.loop(0, NCH // 2)
        def _(t):
            j0 = t * 2
            last = t >= NCH // 2 - 1
            g_wait(0)
            accumulate(0)

            @pl.when(~last)
            def _():
                i_start(j0 + 2, 0)
                i_wait(j0 + 2, 0)
                add_w(0)
                g_start(0)

            g_wait(1)
            accumulate(1)

            @pl.when(~last)
            def _():
                i_start(j0 + 3, 1)
                i_wait(j0 + 3, 1)
                add_w(1)
                g_start(1)

        pltpu.sync_copy(acc, act_h.at[b, :, pl.ds(w * CW, CW)])


def _sc_call(emb32, tok32, posch):
    mesh = plsc.VectorSubcoreMesh(core_axis_name="c", subcore_axis_name="s",
                                  num_cores=NC, num_subcores=NS)
    f = pl.kernel(
        _sc_body,
        out_type=jax.ShapeDtypeStruct((B, SEQ, H), jnp.float32),
        mesh=mesh,
        compiler_params=pltpu.CompilerParams(needs_layout_passes=False,
                                             use_tc_tiling_on_sc=False),
        scratch_types=(
            pltpu.VMEM((2, CK3, CW), jnp.float32),
            pltpu.VMEM((NCH, CK), jnp.int32),
            pltpu.VMEM((SEQ,), jnp.float32),
            pltpu.VMEM((SEQ, CW), jnp.float32),
            pltpu.SemaphoreType.DMA((2,)),
            pltpu.SemaphoreType.DMA((2,)),
            pltpu.VMEM((2, CK3), jnp.int32),
            pltpu.VMEM((2, CK), jnp.int32),
        ),
    )
    return f(emb32, tok32, posch)


def kernel(trigram_set_position_ids, trigram_token_ids,
           trigram_token_ids_offsets, seq_len, emb_weight):
    del trigram_token_ids_offsets, seq_len
    vocab = emb_weight.shape[0]

    emb32 = emb_weight.reshape(vocab * NW, CW)
    tok32 = (trigram_token_ids.astype(jnp.int32) * NW).reshape(B, NCH, CK3)
    posch = trigram_set_position_ids.reshape(B, NCH, CK)

    return _sc_call(emb32, tok32, posch)
